# baseline (speedup 1.0000x reference)
import functools

import jax
import jax.numpy as jnp
from jax import lax
from jax.experimental import pallas as pl
from jax.experimental.pallas import tpu as pltpu
from jax.experimental.pallas import tpu_sc as plsc

_GW = 128
_K = 8


def _make_gather(V: int, D: int, B: int):
    info = plsc.get_sparse_core_info()
    nc, ns = info.num_cores, info.num_subcores
    nw = nc * ns
    rows_per_w = B // nw
    assert rows_per_w % (_K * _GW) == 0
    n_chunks = rows_per_w // (_K * _GW)
    chunk = _K * _GW

    mesh = plsc.VectorSubcoreMesh(core_axis_name="c", subcore_axis_name="s")

    @functools.partial(
        pl.kernel,
        mesh=mesh,
        out_type=jax.ShapeDtypeStruct((B, D), jnp.float32),
        compiler_params=pltpu.CompilerParams(use_tc_tiling_on_sc=False),
        scratch_types=[
            pltpu.VMEM((_K, _GW), jnp.int32),
            pltpu.VMEM((chunk, D), jnp.float32),
            pltpu.SemaphoreType.DMA,
        ],
    )
    def k(table_hbm, idx_hbm, out_hbm, idx_v, rows_v, sem):
        wid = lax.axis_index("s") * nc + lax.axis_index("c")
        idx_row0 = wid * (rows_per_w // _GW)
        out_row0 = wid * rows_per_w

        def body(j, carry):
            pltpu.sync_copy(idx_hbm.at[pl.ds(idx_row0 + j * _K, _K)], idx_v)
            copies = []
            for g in range(_K):
                copies.append(pltpu.async_copy(
                    table_hbm.at[idx_v.at[g]],
                    rows_v.at[pl.ds(g * _GW, _GW)],
                    sem))
            for c in copies:
                c.wait()
            pltpu.sync_copy(rows_v,
                            out_hbm.at[pl.ds(out_row0 + j * chunk, chunk)])
            return carry

        lax.fori_loop(0, n_chunks, body, 0)

    return k


def kernel(x, table):
    bt, s = x.shape
    v, d = table.shape
    b = bt * s
    idx2d = x.reshape(b // _GW, _GW)
    out = _make_gather(v, d, b)(table, idx2d)
    return out.reshape(bt, s, d)


# speedup vs baseline: 1.0142x; 1.0142x over previous
import functools

import jax
import jax.numpy as jnp
from jax import lax
from jax.experimental import pallas as pl
from jax.experimental.pallas import tpu as pltpu
from jax.experimental.pallas import tpu_sc as plsc

_GW = 128
_K = 5
_NBUF = 2


def _make_gather(V: int, D: int, B: int):
    info = plsc.get_sparse_core_info()
    nc, ns = info.num_cores, info.num_subcores
    nw = nc * ns
    rows_per_w = B // nw
    chunk = _K * _GW
    assert rows_per_w % (_NBUF * chunk) == 0
    n_outer = rows_per_w // (_NBUF * chunk)

    mesh = plsc.VectorSubcoreMesh(core_axis_name="c", subcore_axis_name="s")

    @functools.partial(
        pl.kernel,
        mesh=mesh,
        out_type=jax.ShapeDtypeStruct((B, D), jnp.float32),
        compiler_params=pltpu.CompilerParams(use_tc_tiling_on_sc=False),
        scratch_types=[
            pltpu.VMEM((_NBUF * _K, _GW), jnp.int32),
            pltpu.VMEM((_NBUF, chunk, D), jnp.float32),
            pltpu.SemaphoreType.DMA,
            pltpu.SemaphoreType.DMA,
            pltpu.SemaphoreType.DMA,
            pltpu.SemaphoreType.DMA,
        ],
    )
    def k(table_hbm, idx_hbm, out_hbm, idx_v, rows_v, gsem0, gsem1,
          wsem0, wsem1):
        wid = lax.axis_index("s") * nc + lax.axis_index("c")
        idx_row0 = wid * (rows_per_w // _GW)
        out_row0 = wid * rows_per_w
        gsems = (gsem0, gsem1)
        wsems = (wsem0, wsem1)

        def body(t, carry):
            pltpu.sync_copy(
                idx_hbm.at[pl.ds(idx_row0 + t * _NBUF * _K, _NBUF * _K)],
                idx_v)
            handles = []
            for b in range(_NBUF):
                j = t * _NBUF + b

                @pl.when(t > 0)
                def _drain_prev_write(b=b, j=j):
                    pltpu.make_async_copy(
                        rows_v.at[b],
                        out_hbm.at[pl.ds(out_row0 + (j - _NBUF) * chunk,
                                         chunk)],
                        wsems[b]).wait()

                hs = []
                for g in range(_K):
                    hs.append(pltpu.async_copy(
                        table_hbm.at[idx_v.at[b * _K + g]],
                        rows_v.at[b].at[pl.ds(g * _GW, _GW)],
                        gsems[b]))
                handles.append(hs)
            for b in range(_NBUF):
                j = t * _NBUF + b
                for h in handles[b]:
                    h.wait()
                pltpu.async_copy(
                    rows_v.at[b],
                    out_hbm.at[pl.ds(out_row0 + j * chunk, chunk)],
                    wsems[b])
            return carry

        lax.fori_loop(0, n_outer, body, 0)
        for b in range(_NBUF):
            j = (n_outer - 1) * _NBUF + b
            pltpu.make_async_copy(
                rows_v.at[b],
                out_hbm.at[pl.ds(out_row0 + j * chunk, chunk)],
                wsems[b]).wait()

    return k


def kernel(x, table):
    bt, s = x.shape
    v, d = table.shape
    b = bt * s
    idx2d = x.reshape(b // _GW, _GW)
    out = _make_gather(v, d, b)(table, idx2d)
    return out.reshape(bt, s, d)


# speedup vs baseline: 1.0164x; 1.0022x over previous
import functools

import jax
import jax.numpy as jnp
from jax import lax
from jax.experimental import pallas as pl
from jax.experimental.pallas import tpu as pltpu
from jax.experimental.pallas import tpu_sc as plsc

_GW = 128
_K = 5
_NBUF = 2


def _make_gather(V: int, D: int, B: int):
    info = plsc.get_sparse_core_info()
    nc, ns = info.num_cores, info.num_subcores
    nw = nc * ns
    rows_per_w = B // nw
    chunk = _K * _GW
    assert rows_per_w % (_NBUF * chunk) == 0
    n_outer = rows_per_w // (_NBUF * chunk)

    mesh = plsc.VectorSubcoreMesh(core_axis_name="c", subcore_axis_name="s")

    @functools.partial(
        pl.kernel,
        mesh=mesh,
        out_type=jax.ShapeDtypeStruct((B, D), jnp.float32),
        compiler_params=pltpu.CompilerParams(use_tc_tiling_on_sc=False),
        scratch_types=[
            pltpu.VMEM((_NBUF, _K * _GW), jnp.int32),
            pltpu.VMEM((_NBUF, chunk, D), jnp.float32),
            pltpu.SemaphoreType.DMA,
            pltpu.SemaphoreType.DMA,
            pltpu.SemaphoreType.DMA,
            pltpu.SemaphoreType.DMA,
        ],
    )
    def k(table_hbm, idx_hbm, out_hbm, idx_v, rows_v, gsem0, gsem1,
          wsem0, wsem1):
        wid = lax.axis_index("s") * nc + lax.axis_index("c")
        idx0 = wid * rows_per_w
        out_row0 = wid * rows_per_w
        gsems = (gsem0, gsem1)
        wsems = (wsem0, wsem1)

        def body(t, carry):
            handles = []
            for b in range(_NBUF):
                j = t * _NBUF + b
                pltpu.sync_copy(
                    idx_hbm.at[pl.ds(idx0 + j * chunk, chunk)],
                    idx_v.at[b])

                @pl.when(t > 0)
                def _drain_prev_write(b=b, j=j):
                    pltpu.make_async_copy(
                        rows_v.at[b],
                        out_hbm.at[pl.ds(out_row0 + (j - _NBUF) * chunk,
                                         chunk)],
                        wsems[b]).wait()

                handles.append([pltpu.async_copy(
                    table_hbm.at[idx_v.at[b]],
                    rows_v.at[b],
                    gsems[b])])
            for b in range(_NBUF):
                j = t * _NBUF + b
                for h in handles[b]:
                    h.wait()
                pltpu.async_copy(
                    rows_v.at[b],
                    out_hbm.at[pl.ds(out_row0 + j * chunk, chunk)],
                    wsems[b])
            return carry

        lax.fori_loop(0, n_outer, body, 0)
        for b in range(_NBUF):
            j = (n_outer - 1) * _NBUF + b
            pltpu.make_async_copy(
                rows_v.at[b],
                out_hbm.at[pl.ds(out_row0 + j * chunk, chunk)],
                wsems[b]).wait()

    return k


def kernel(x, table):
    bt, s = x.shape
    v, d = table.shape
    b = bt * s
    idx = x.reshape(b)
    out = _make_gather(v, d, b)(table, idx)
    return out.reshape(bt, s, d)
